# baseline (speedup 1.0000x reference)
import functools

import jax
import jax.numpy as jnp
from jax import lax
from jax.experimental import pallas as pl
from jax.experimental.pallas import tpu as pltpu
from jax.experimental.pallas import tpu_sc as plsc

N_NODES = 10000
N_PAIRS = 320000
HIDDEN = 128
VOCAB = 22
N_CYCLES = 10000

STRIPE = 2560
HSIZE = STRIPE * 128
NC, NS = 2, 16
CHUNK = N_PAIRS // (NC * NS)
ROWS = CHUNK // 16
HTILE = HSIZE // NS
ZCH = 2048
SLAB = 10240
SCCH = 2000
WCH = 5120


def _sc_hist_body(atc_hbm, x_hbm, c_hbm,
                  x_v, atc_v, idx_a, idx_b, ones_v, zero_v, wb_v, hist_sh,
                  in_sem, z_sem, s_sem):
    cid = lax.axis_index("c")
    s = lax.axis_index("s")
    w = cid * NS + s
    base = w * CHUNK

    start = jnp.minimum(base - base % 128, N_PAIRS - SLAB)
    start = pl.multiple_of(start, 128)
    off = base - start
    cp_x = pltpu.async_copy(x_hbm, x_v, in_sem)
    cp_a = pltpu.async_copy(atc_hbm.at[:, pl.ds(start, SLAB)], atc_v, in_sem)

    @plsc.parallel_loop(0, ZCH, step=16, unroll=8)
    def zloop(i):
        zero_v[pl.ds(i, 16)] = jnp.zeros((16,), jnp.float32)
    zcps = [
        pltpu.async_copy(zero_v, hist_sh.at[pl.ds(s * HTILE + j * ZCH, ZCH)],
                         z_sem)
        for j in range(HTILE // ZCH)
    ]

    ones16 = jnp.ones((16,), jnp.float32)
    @plsc.parallel_loop(0, SCCH, step=16, unroll=8)
    def oloop(i):
        ones_v[pl.ds(i, 16)] = ones16

    cp_x.wait()
    cp_a.wait()

    idx_bufs = (idx_a, idx_b)

    def build(j, buf):
        @plsc.parallel_loop(0, SCCH, step=16, unroll=4)
        def iloop(i):
            a0_16 = atc_v[0, pl.ds(off + j * SCCH + i, 16)]
            v16 = plsc.load_gather(x_v, [a0_16])
            c16 = atc_v[1, pl.ds(off + j * SCCH + i, 16)]
            g16 = (c16 * 26215) >> 26
            r16 = c16 - ((g16 << 11) + (g16 << 9))
            buf[pl.ds(i, 16)] = (r16 << 7) | (g16 << 5) | v16

    build(0, idx_bufs[0])
    for cp in zcps:
        cp.wait()
    plsc.subcore_barrier()
    scps = []
    nchunks = CHUNK // SCCH
    for j in range(nchunks):
        scps.append(pltpu.async_copy(
            ones_v, hist_sh.at[idx_bufs[j % 2]], s_sem, add=True))
        if j + 1 < nchunks:
            if j >= 1:
                scps[j - 1].wait()
            build(j + 1, idx_bufs[(j + 1) % 2])
    scps[nchunks - 2].wait()
    scps[nchunks - 1].wait()
    plsc.subcore_barrier()

    wcps = []
    for k in range(HTILE // WCH):
        pltpu.async_copy(
            hist_sh.at[pl.ds(s * HTILE + k * WCH, WCH)],
            wb_v.at[pl.ds(k * WCH, WCH)], z_sem).wait()
        wcps.append(pltpu.async_copy(
            wb_v.at[pl.ds(k * WCH, WCH)],
            c_hbm.at[pl.ds(cid * HSIZE + s * HTILE + k * WCH, WCH)], s_sem))
    for cp in wcps:
        cp.wait()


_sc_hist = functools.partial(
    pl.kernel,
    out_type=jax.ShapeDtypeStruct((NC * HSIZE,), jnp.float32),
    mesh=plsc.VectorSubcoreMesh(core_axis_name="c", subcore_axis_name="s"),
    scratch_types=[
        pltpu.VMEM((N_NODES,), jnp.int32),
        pltpu.VMEM((2, SLAB), jnp.int32),
        pltpu.VMEM((SCCH,), jnp.int32),
        pltpu.VMEM((SCCH,), jnp.int32),
        pltpu.VMEM((SCCH,), jnp.float32),
        pltpu.VMEM((ZCH,), jnp.float32),
        pltpu.VMEM((HTILE,), jnp.float32),
        pltpu.VMEM_SHARED((HSIZE,), jnp.float32),
        pltpu.SemaphoreType.DMA,
        pltpu.SemaphoreType.DMA,
        pltpu.SemaphoreType.DMA,
    ],
    compiler_params=pltpu.CompilerParams(needs_layout_passes=False),
)(_sc_hist_body)


def _tc_mm_body(c0_ref, c1_ref, e_ref, o_ref):
    o_ref[...] = jnp.dot(c0_ref[...] + c1_ref[...], e_ref[0],
                         preferred_element_type=jnp.float32)


def kernel(x, atom_to_cycle, emb_table):
    c = _sc_hist(atom_to_cycle, x)
    c2d = c.reshape(NC * STRIPE, 128)
    e4 = jnp.zeros((4, 128, HIDDEN), emb_table.dtype)
    for g in range(4):
        e4 = e4.at[g, g * 32:g * 32 + VOCAB, :].set(emb_table)
    out = pl.pallas_call(
        _tc_mm_body,
        grid=(4,),
        in_specs=[
            pl.BlockSpec((STRIPE, 128), lambda g: (0, 0)),
            pl.BlockSpec((STRIPE, 128), lambda g: (1, 0)),
            pl.BlockSpec((1, 128, HIDDEN), lambda g: (g, 0, 0)),
        ],
        out_specs=pl.BlockSpec((STRIPE, HIDDEN), lambda g: (g, 0)),
        out_shape=jax.ShapeDtypeStruct((N_CYCLES, HIDDEN), jnp.float32),
    )(c2d, c2d, e4)
    return out


# speedup vs baseline: 1.0089x; 1.0089x over previous
import functools

import jax
import jax.numpy as jnp
from jax import lax
from jax.experimental import pallas as pl
from jax.experimental.pallas import tpu as pltpu
from jax.experimental.pallas import tpu_sc as plsc

N_NODES = 10000
N_PAIRS = 320000
HIDDEN = 128
VOCAB = 22
N_CYCLES = 10000

STRIPE = 2560
HSIZE = STRIPE * 128
NC, NS = 2, 16
CHUNK = N_PAIRS // (NC * NS)
ROWS = CHUNK // 16
HTILE = HSIZE // NS
ZCH = 4096
SLAB = 10240
SCCH = 2000
WCH = 5120


def _sc_hist_body(atc_hbm, x_hbm, c_hbm,
                  x_v, atc_v, idx_a, idx_b, ones_v, zero_v, wb_v, hist_sh,
                  in_sem, z_sem, s_sem):
    cid = lax.axis_index("c")
    s = lax.axis_index("s")
    w = cid * NS + s
    base = w * CHUNK

    start = jnp.minimum(base - base % 128, N_PAIRS - SLAB)
    start = pl.multiple_of(start, 128)
    off = base - start
    cp_x = pltpu.async_copy(x_hbm, x_v, in_sem)
    cp_a = pltpu.async_copy(atc_hbm.at[:, pl.ds(start, SLAB)], atc_v, in_sem)

    @plsc.parallel_loop(0, ZCH, step=16, unroll=4)
    def zloop(i):
        zero_v[pl.ds(i, 16)] = jnp.zeros((16,), jnp.float32)
    zcps = [
        pltpu.async_copy(zero_v, hist_sh.at[pl.ds(s * HTILE + j * ZCH, ZCH)],
                         z_sem)
        for j in range(HTILE // ZCH)
    ]

    ones16 = jnp.ones((16,), jnp.float32)
    @plsc.parallel_loop(0, SCCH, step=16, unroll=4)
    def oloop(i):
        ones_v[pl.ds(i, 16)] = ones16

    cp_x.wait()
    cp_a.wait()

    idx_bufs = (idx_a, idx_b)

    def build(j, buf):
        @plsc.parallel_loop(0, SCCH, step=16, unroll=2)
        def iloop(i):
            a0_16 = atc_v[0, pl.ds(off + j * SCCH + i, 16)]
            v16 = plsc.load_gather(x_v, [a0_16])
            c16 = atc_v[1, pl.ds(off + j * SCCH + i, 16)]
            g16 = (c16 * 26215) >> 26
            r16 = c16 - ((g16 << 11) + (g16 << 9))
            buf[pl.ds(i, 16)] = (r16 << 7) | (g16 << 5) | v16

    build(0, idx_bufs[0])
    for cp in zcps:
        cp.wait()
    plsc.subcore_barrier()
    scps = []
    nchunks = CHUNK // SCCH
    for j in range(nchunks):
        scps.append(pltpu.async_copy(
            ones_v, hist_sh.at[idx_bufs[j % 2]], s_sem, add=True))
        if j + 1 < nchunks:
            if j >= 1:
                scps[j - 1].wait()
            build(j + 1, idx_bufs[(j + 1) % 2])
    scps[nchunks - 2].wait()
    scps[nchunks - 1].wait()
    plsc.subcore_barrier()

    wcps = []
    for k in range(HTILE // WCH):
        pltpu.async_copy(
            hist_sh.at[pl.ds(s * HTILE + k * WCH, WCH)],
            wb_v.at[pl.ds(k * WCH, WCH)], z_sem).wait()
        wcps.append(pltpu.async_copy(
            wb_v.at[pl.ds(k * WCH, WCH)],
            c_hbm.at[pl.ds(cid * HSIZE + s * HTILE + k * WCH, WCH)], s_sem))
    for cp in wcps:
        cp.wait()


_sc_hist = functools.partial(
    pl.kernel,
    out_type=jax.ShapeDtypeStruct((NC * HSIZE,), jnp.float32),
    mesh=plsc.VectorSubcoreMesh(core_axis_name="c", subcore_axis_name="s"),
    scratch_types=[
        pltpu.VMEM((N_NODES,), jnp.int32),
        pltpu.VMEM((2, SLAB), jnp.int32),
        pltpu.VMEM((SCCH,), jnp.int32),
        pltpu.VMEM((SCCH,), jnp.int32),
        pltpu.VMEM((SCCH,), jnp.float32),
        pltpu.VMEM((ZCH,), jnp.float32),
        pltpu.VMEM((HTILE,), jnp.float32),
        pltpu.VMEM_SHARED((HSIZE,), jnp.float32),
        pltpu.SemaphoreType.DMA,
        pltpu.SemaphoreType.DMA,
        pltpu.SemaphoreType.DMA,
    ],
    compiler_params=pltpu.CompilerParams(needs_layout_passes=False),
)(_sc_hist_body)


def _tc_mm_body(c0_ref, c1_ref, e_ref, o_ref):
    o_ref[...] = jnp.dot(c0_ref[...] + c1_ref[...], e_ref[0],
                         preferred_element_type=jnp.float32)


def kernel(x, atom_to_cycle, emb_table):
    c = _sc_hist(atom_to_cycle, x)
    c2d = c.reshape(NC * STRIPE, 128)
    e4 = jnp.zeros((4, 128, HIDDEN), emb_table.dtype)
    for g in range(4):
        e4 = e4.at[g, g * 32:g * 32 + VOCAB, :].set(emb_table)
    out = pl.pallas_call(
        _tc_mm_body,
        grid=(4,),
        in_specs=[
            pl.BlockSpec((STRIPE, 128), lambda g: (0, 0)),
            pl.BlockSpec((STRIPE, 128), lambda g: (1, 0)),
            pl.BlockSpec((1, 128, HIDDEN), lambda g: (g, 0, 0)),
        ],
        out_specs=pl.BlockSpec((STRIPE, HIDDEN), lambda g: (g, 0)),
        out_shape=jax.ShapeDtypeStruct((N_CYCLES, HIDDEN), jnp.float32),
    )(c2d, c2d, e4)
    return out
